# Initial kernel scaffold; baseline (speedup 1.0000x reference)
import jax
import jax.numpy as jnp
from jax.experimental import pallas as pl


def kernel(input, table):
    raise NotImplementedError("write your pallas kernel here")


# speedup vs baseline: 1.4845x; 1.4845x over previous
import functools
import math

import jax
import jax.numpy as jnp
import numpy as np
from jax import lax
from jax.experimental import pallas as pl
from jax.experimental.pallas import tpu as pltpu
from jax.experimental.pallas import tpu_sc as plsc


def _positional_embedding(num_positions, m):
    pos = jnp.arange(num_positions, dtype=jnp.float32)
    exp = jnp.arange(m, dtype=jnp.float32) / m * jnp.log2(jnp.float32(10000.0))
    denom = jnp.exp2(exp)
    arg = pos[:, None] / denom[None, :]
    even = (jnp.arange(m) % 2) == 0
    return jnp.where(even[None, :], jnp.sin(arg), jnp.cos(arg))


@functools.partial(jax.jit, static_argnames=("ch",))
def _sc_embed(idx2, table, pos, *, ch):
    n_chunks = idx2.shape[0]
    t, m = table.shape
    c = pos.shape[0]
    info = plsc.get_sparse_core_info()
    nc, ns = info.num_cores, info.num_subcores
    nw = nc * ns
    cpw = n_chunks // nw
    scale = np.sqrt(np.float32(t)).astype(np.float32)
    mesh = plsc.VectorSubcoreMesh(core_axis_name="c", subcore_axis_name="s")

    @functools.partial(
        pl.kernel,
        mesh=mesh,
        out_type=jax.ShapeDtypeStruct((n_chunks, ch, m), jnp.float32),
        scratch_types=[
            pltpu.VMEM((cpw, ch), jnp.int32),
            pltpu.VMEM((c, m), jnp.float32),
            pltpu.VMEM((ch, m), jnp.float32),
            pltpu.SemaphoreType.DMA,
        ],
    )
    def k(idx_hbm, table_hbm, pos_hbm, out_hbm, idx_v, pos_v, rows_v, sem):
        wid = lax.axis_index("s") * nc + lax.axis_index("c")
        base = wid * cpw
        pltpu.sync_copy(idx_hbm.at[pl.ds(base, cpw)], idx_v)
        pltpu.sync_copy(pos_hbm, pos_v)

        def chunk_body(i, carry):
            cidx = base + i
            pltpu.async_copy(table_hbm.at[idx_v.at[i]], rows_v, sem).wait()
            poff = (cidx % (c // ch)) * ch

            def j_body(j, carry2):
                pj = poff + j
                for l in range(m // 16):
                    sl = pl.ds(l * 16, 16)
                    rows_v[j, sl] = rows_v[j, sl] * scale + pos_v[pj, sl]
                return carry2

            lax.fori_loop(0, ch, j_body, 0, unroll=1)
            pltpu.sync_copy(rows_v, out_hbm.at[cidx])
            return carry

        lax.fori_loop(0, cpw, chunk_body, 0, unroll=1)

    return k(idx2, table, pos)


def kernel(input, table):
    b, c = input.shape
    t, m = table.shape
    ch = 100
    pos = _positional_embedding(c, m)
    idx2 = input.astype(jnp.int32).reshape(b * c // ch, ch)
    out = _sc_embed(idx2, table, pos, ch=ch)
    return out.reshape(b, c, m)


# speedup vs baseline: 2.0905x; 1.4082x over previous
import functools

import jax
import jax.numpy as jnp
import numpy as np
from jax import lax
from jax.experimental import pallas as pl
from jax.experimental.pallas import tpu as pltpu
from jax.experimental.pallas import tpu_sc as plsc


def _positional_embedding(num_positions, m):
    pos = jnp.arange(num_positions, dtype=jnp.float32)
    exp = jnp.arange(m, dtype=jnp.float32) / m * jnp.log2(jnp.float32(10000.0))
    denom = jnp.exp2(exp)
    arg = pos[:, None] / denom[None, :]
    even = (jnp.arange(m) % 2) == 0
    return jnp.where(even[None, :], jnp.sin(arg), jnp.cos(arg))


@functools.partial(jax.jit, static_argnames=("ch",))
def _sc_embed(idx2, table, pos, *, ch):
    n_chunks = idx2.shape[0]
    t, m = table.shape
    c = pos.shape[0]
    info = plsc.get_sparse_core_info()
    nc, ns = info.num_cores, info.num_subcores
    nw = nc * ns
    cpw = n_chunks // nw
    n_groups = (cpw + 2) // 3
    scale = np.sqrt(np.float32(t)).astype(np.float32)
    mesh = plsc.VectorSubcoreMesh(core_axis_name="c", subcore_axis_name="s")

    @functools.partial(
        pl.kernel,
        mesh=mesh,
        out_type=jax.ShapeDtypeStruct((n_chunks, ch, m), jnp.float32),
        scratch_types=[
            pltpu.VMEM((cpw, ch), jnp.int32),
            pltpu.VMEM((c, m), jnp.float32),
            pltpu.VMEM((ch, m), jnp.float32),
            pltpu.VMEM((ch, m), jnp.float32),
            pltpu.VMEM((ch, m), jnp.float32),
            pltpu.SemaphoreType.DMA,
            pltpu.SemaphoreType.DMA,
            pltpu.SemaphoreType.DMA,
            pltpu.SemaphoreType.DMA,
            pltpu.SemaphoreType.DMA,
            pltpu.SemaphoreType.DMA,
        ],
    )
    def k(idx_hbm, table_hbm, pos_hbm, out_hbm, idx_v, pos_v,
          rows0, rows1, rows2, g0, g1, g2, s0, s1, s2):
        rows = (rows0, rows1, rows2)
        gsem = (g0, g1, g2)
        ssem = (s0, s1, s2)
        wid = lax.axis_index("s") * nc + lax.axis_index("c")
        base = wid * cpw
        pltpu.sync_copy(idx_hbm.at[pl.ds(base, cpw)], idx_v)
        pltpu.sync_copy(pos_hbm, pos_v)
        pltpu.async_copy(table_hbm.at[idx_v.at[0]], rows0, g0)
        pltpu.async_copy(table_hbm.at[idx_v.at[1]], rows1, g1)

        def slot(g, b):
            rv, nv = rows[b], rows[(b + 2) % 3]
            pltpu.make_async_copy(table_hbm.at[idx_v.at[g]], rv, gsem[b]).wait()
            poff = ((base + g) % (c // ch)) * ch

            def j_body(j, carry2):
                pj = poff + j
                for l in range(m // 16):
                    sl = pl.ds(l * 16, 16)
                    rv[j, sl] = rv[j, sl] * scale + pos_v[pj, sl]
                return carry2

            lax.fori_loop(0, ch, j_body, 0, unroll=2)
            pltpu.async_copy(rv, out_hbm.at[base + g], ssem[b])

            @pl.when(g >= 1)
            def _():
                pltpu.make_async_copy(
                    nv, out_hbm.at[base + g - 1], ssem[(b + 2) % 3]).wait()

            @pl.when(g + 2 < cpw)
            def _():
                pltpu.async_copy(
                    table_hbm.at[idx_v.at[g + 2]], nv, gsem[(b + 2) % 3])

        def group(p, carry):
            for b in range(3):
                g = p * 3 + b

                @pl.when(g < cpw)
                def _():
                    slot(g, b)
            return carry

        lax.fori_loop(0, n_groups, group, 0, unroll=1)
        pltpu.make_async_copy(
            rows[(cpw - 1) % 3], out_hbm.at[base + cpw - 1],
            ssem[(cpw - 1) % 3]).wait()

    return k(idx2, table, pos)


def kernel(input, table):
    b, c = input.shape
    t, m = table.shape
    ch = 100
    pos = _positional_embedding(c, m)
    idx2 = input.astype(jnp.int32).reshape(b * c // ch, ch)
    out = _sc_embed(idx2, table, pos, ch=ch)
    return out.reshape(b, c, m)
